# baseline (speedup 1.0000x reference)
import functools

import jax
import jax.numpy as jnp
from jax import lax
from jax.experimental import pallas as pl
from jax.experimental.pallas import tpu as pltpu
from jax.experimental.pallas import tpu_sc as plsc

_VOCAB = 100000
_DIM = 64
_NC = 2
_NS = 16
_NW = _NC * _NS
_CH = 400
_C = _VOCAB // _CH
_FULL = _C // _NW
_EXTRA = _C - _FULL * _NW

_MESH = plsc.VectorSubcoreMesh(core_axis_name="c", subcore_axis_name="s")


@functools.partial(
    pl.kernel,
    out_type=jax.ShapeDtypeStruct((_VOCAB, _DIM), jnp.float32),
    mesh=_MESH,
    scratch_types=[pltpu.SemaphoreType.DMA],
)
def _sc_copy(x_hbm, o_hbm, sem):
    w = lax.axis_index("s") * _NC + lax.axis_index("c")

    def chunk_copy(j):
        r = pl.ds(pl.multiple_of((w + _NW * j) * _CH, 8), _CH)
        return pltpu.make_async_copy(x_hbm.at[r, :], o_hbm.at[r, :], sem)

    for j in range(_FULL):
        chunk_copy(j).start()

    @pl.when(w < _EXTRA)
    def _():
        chunk_copy(_FULL).start()
        chunk_copy(_FULL).wait()

    for j in range(_FULL):
        chunk_copy(j).wait()


def kernel(embeddings):
    return _sc_copy(embeddings)


# speedup vs baseline: 8.3447x; 8.3447x over previous
import functools

import jax
import jax.numpy as jnp
from jax import lax
from jax.experimental import pallas as pl
from jax.experimental.pallas import tpu as pltpu
from jax.experimental.pallas import tpu_sc as plsc

_VOCAB = 100000
_DIM = 64
_NC = 2
_NS = 16
_NW = _NC * _NS
_CH = 400

_SC_ROWS = 45200
_TC_ROWS = _VOCAB - _SC_ROWS
_C = _SC_ROWS // _CH
_FULL = _C // _NW
_EXTRA = _C - _FULL * _NW
_TC_BLOCKS = _TC_ROWS // _CH

_MESH = plsc.VectorSubcoreMesh(core_axis_name="c", subcore_axis_name="s")


@functools.partial(
    pl.kernel,
    out_type=jax.ShapeDtypeStruct((_SC_ROWS, _DIM), jnp.float32),
    mesh=_MESH,
    scratch_types=[
        pltpu.VMEM((_CH, _DIM), jnp.float32),
        pltpu.VMEM((_CH, _DIM), jnp.float32),
        pltpu.SemaphoreType.DMA,
        pltpu.SemaphoreType.DMA,
        pltpu.SemaphoreType.DMA,
        pltpu.SemaphoreType.DMA,
    ],
)
def _sc_copy(x_hbm, o_hbm, buf_a, buf_b, ls_a, ls_b, ss_a, ss_b):
    w = lax.axis_index("s") * _NC + lax.axis_index("c")
    bufs = (buf_a, buf_b)
    lsem = (ls_a, ls_b)
    ssem = (ss_a, ss_b)

    def rows(j):
        return pl.ds(pl.multiple_of((w + _NW * j) * _CH, 8), _CH)

    def load(j):
        return pltpu.make_async_copy(x_hbm.at[rows(j), :], bufs[j % 2], lsem[j % 2])

    def store(j):
        return pltpu.make_async_copy(bufs[j % 2], o_hbm.at[rows(j), :], ssem[j % 2])

    load(0).start()
    for j in range(_FULL):
        load(j).wait()
        store(j).start()
        if j + 1 < _FULL:
            if j >= 1:
                store(j - 1).wait()
            load(j + 1).start()

    @pl.when(w < _EXTRA)
    def _():
        j = _FULL
        store(j - 2).wait()
        load(j).start()
        load(j).wait()
        store(j).start()
        store(j).wait()

    @pl.when(w >= _EXTRA)
    def _():
        store(_FULL - 2).wait()

    store(_FULL - 1).wait()


def _tc_body(x_ref, o_ref):
    o_ref[...] = x_ref[...]


def _tc_copy(x):
    return pl.pallas_call(
        _tc_body,
        grid=(_TC_BLOCKS,),
        in_specs=[pl.BlockSpec((_CH, _DIM), lambda i: (i + _C, 0))],
        out_specs=pl.BlockSpec((_CH, _DIM), lambda i: (i, 0)),
        out_shape=jax.ShapeDtypeStruct((_TC_ROWS, _DIM), jnp.float32),
    )(x)


def kernel(embeddings):
    head = _sc_copy(embeddings)
    tail = _tc_copy(embeddings)
    return jnp.concatenate([head, tail], axis=0)


# speedup vs baseline: 13.3346x; 1.5980x over previous
import functools

import jax
import jax.numpy as jnp
from jax import lax
from jax.experimental import pallas as pl
from jax.experimental.pallas import tpu as pltpu
from jax.experimental.pallas import tpu_sc as plsc

_VOCAB = 100000
_DIM = 64
_NC = 2
_NS = 16
_NW = _NC * _NS
_CH = 200
_C = _VOCAB // _CH
_FULL = _C // _NW
_EXTRA = _C - _FULL * _NW

_MESH = plsc.VectorSubcoreMesh(core_axis_name="c", subcore_axis_name="s")


@functools.partial(
    pl.kernel,
    out_type=jax.ShapeDtypeStruct((_VOCAB, _DIM), jnp.float32),
    mesh=_MESH,
    scratch_types=[
        pltpu.VMEM_SHARED((_NS, 2, _CH, _DIM), jnp.float32),
        pltpu.SemaphoreType.DMA,
        pltpu.SemaphoreType.DMA,
        pltpu.SemaphoreType.DMA,
        pltpu.SemaphoreType.DMA,
    ],
)
def _sc_copy(x_hbm, o_hbm, shared, ls_a, ls_b, ss_a, ss_b):
    s = lax.axis_index("s")
    w = s * _NC + lax.axis_index("c")
    lsem = (ls_a, ls_b)
    ssem = (ss_a, ss_b)

    def rows(j):
        return pl.ds(pl.multiple_of((w + _NW * j) * _CH, 8), _CH)

    def load(j):
        return pltpu.make_async_copy(
            x_hbm.at[rows(j), :], shared.at[s, j % 2], lsem[j % 2])

    def store(j):
        return pltpu.make_async_copy(
            shared.at[s, j % 2], o_hbm.at[rows(j), :], ssem[j % 2])

    load(0).start()
    for j in range(_FULL):
        load(j).wait()
        store(j).start()
        if j + 1 < _FULL:
            if j >= 1:
                store(j - 1).wait()
            load(j + 1).start()

    @pl.when(w < _EXTRA)
    def _():
        j = _FULL
        store(j - 2).wait()
        load(j).start()
        load(j).wait()
        store(j).start()
        store(j).wait()

    @pl.when(w >= _EXTRA)
    def _():
        store(_FULL - 2).wait()

    store(_FULL - 1).wait()


def kernel(embeddings):
    return _sc_copy(embeddings)
